# Initial kernel scaffold; baseline (speedup 1.0000x reference)
import jax
import jax.numpy as jnp
from jax.experimental import pallas as pl


def kernel(x, edge_index, W1, b1, W2, b2):
    raise NotImplementedError("write your pallas kernel here")


# speedup vs baseline: 29.3392x; 29.3392x over previous
import functools

import jax
import jax.numpy as jnp
from jax import lax
from jax.experimental import pallas as pl
from jax.experimental.pallas import tpu as pltpu
from jax.experimental.pallas import tpu_sc as plsc

N = 10000
D = 128
E = 320000
NP = 10240
NW = 32
K = 80
NBLK = 128
EPW = K * NBLK
EP = NW * EPW
RPT = NP // 16

def _mesh():
    return plsc.VectorSubcoreMesh(
        core_axis_name="c", subcore_axis_name="s", num_cores=2,
        num_subcores=16)


def _deg_body(dst_hbm, zeros1_hbm, ones_hbm, out_hbm, didx, ones_v, deg_sp):
    c = lax.axis_index("c")
    s = lax.axis_index("s")
    w = s * 2 + c
    pltpu.sync_copy(dst_hbm.at[w], didx)
    pltpu.sync_copy(ones_hbm, ones_v)
    pltpu.sync_copy(zeros1_hbm, deg_sp.at[pl.ds(s * RPT, RPT)])
    plsc.subcore_barrier()

    def body(j, carry):
        pltpu.sync_copy(ones_v, deg_sp.at[didx.at[j]], add=True)
        return carry

    lax.fori_loop(0, NBLK, body, 0)
    plsc.subcore_barrier()
    pltpu.sync_copy(deg_sp.at[pl.ds(s * RPT, RPT)],
                    out_hbm.at[c, pl.ds(s * RPT, RPT)])


@functools.lru_cache(maxsize=None)
def _deg_kernel():
    return pl.kernel(
        _deg_body,
        out_type=jax.ShapeDtypeStruct((2, NP), jnp.float32),
        mesh=_mesh(),
        scratch_types=[
            pltpu.VMEM((NBLK, K), jnp.int32),
            pltpu.VMEM((K,), jnp.float32),
            pltpu.VMEM_SHARED((NP,), jnp.float32),
        ],
    )


def _edge_body(y_hbm, src_hbm, dst_hbm, zrows_hbm, out_hbm,
               sidx, didx, bufs, acc_sp, sem0, sem1):
    c = lax.axis_index("c")
    s = lax.axis_index("s")
    w = s * 2 + c
    sems = (sem0, sem1)
    pltpu.sync_copy(src_hbm.at[w], sidx)
    pltpu.sync_copy(dst_hbm.at[w], didx)
    pltpu.sync_copy(zrows_hbm, acc_sp.at[pl.ds(s * RPT, RPT)])
    plsc.subcore_barrier()

    pltpu.async_copy(y_hbm.at[sidx.at[pl.ds(0, K)]], bufs.at[0], sems[0])
    pltpu.async_copy(y_hbm.at[sidx.at[pl.ds(K, K)]], bufs.at[1], sems[1])

    def body(j, carry):
        for b in range(2):
            jb = j * 2 + b
            pltpu.make_async_copy(y_hbm.at[sidx.at[pl.ds(jb * K, K)]],
                                  bufs.at[b], sems[b]).wait()
            pltpu.sync_copy(bufs.at[b], acc_sp.at[didx.at[jb]], add=True)

            @pl.when(jb + 2 < NBLK)
            def _start():
                pltpu.async_copy(y_hbm.at[sidx.at[pl.ds((jb + 2) * K, K)]],
                                 bufs.at[b], sems[b])
        return carry

    lax.fori_loop(0, NBLK // 2, body, 0)
    plsc.subcore_barrier()
    pltpu.sync_copy(acc_sp.at[pl.ds(s * RPT, RPT)],
                    out_hbm.at[c, pl.ds(s * RPT, RPT)])


@functools.lru_cache(maxsize=None)
def _edge_kernel():
    return pl.kernel(
        _edge_body,
        out_type=jax.ShapeDtypeStruct((2, NP, D), jnp.float32),
        mesh=_mesh(),
        scratch_types=[
            pltpu.VMEM((EPW,), jnp.int32),
            pltpu.VMEM((NBLK, K), jnp.int32),
            pltpu.VMEM((2, K, D), jnp.float32),
            pltpu.VMEM_SHARED((NP, D), jnp.float32),
            pltpu.SemaphoreType.DMA,
            pltpu.SemaphoreType.DMA,
        ],
    )


_BM = 1024
_GRID = NP // _BM


def _dinv_col(deg_ref):
    deg = deg_ref[...]
    ones = jnp.ones((2, 1), jnp.float32)
    degc = lax.dot_general(deg, ones, (((0,), (0,)), ((), ())))
    return 1.0 / jnp.sqrt(degc + 1.0)


def _mm_scale_body(deg_ref, x_ref, w_ref, o_ref):
    dinv = _dinv_col(deg_ref)
    xw = jnp.dot(x_ref[...], w_ref[...], preferred_element_type=jnp.float32)
    o_ref[...] = xw * dinv


_mm_scale = pl.pallas_call(
    _mm_scale_body,
    grid=(_GRID,),
    in_specs=[
        pl.BlockSpec((2, _BM), lambda i: (0, i)),
        pl.BlockSpec((_BM, D), lambda i: (i, 0)),
        pl.BlockSpec((D, D), lambda i: (0, 0)),
    ],
    out_specs=pl.BlockSpec((_BM, D), lambda i: (i, 0)),
    out_shape=jax.ShapeDtypeStruct((NP, D), jnp.float32),
)


def _mid_body(deg_ref, a_ref, y_ref, b_ref, w_ref, o_ref):
    dinv = _dinv_col(deg_ref)
    h = (a_ref[0] + a_ref[1] + y_ref[...]) * dinv + b_ref[...]
    o_ref[...] = jnp.dot(h, w_ref[...],
                         preferred_element_type=jnp.float32) * dinv


_mid = pl.pallas_call(
    _mid_body,
    grid=(_GRID,),
    in_specs=[
        pl.BlockSpec((2, _BM), lambda i: (0, i)),
        pl.BlockSpec((2, _BM, D), lambda i: (0, i, 0)),
        pl.BlockSpec((_BM, D), lambda i: (i, 0)),
        pl.BlockSpec((1, D), lambda i: (0, 0)),
        pl.BlockSpec((D, D), lambda i: (0, 0)),
    ],
    out_specs=pl.BlockSpec((_BM, D), lambda i: (i, 0)),
    out_shape=jax.ShapeDtypeStruct((NP, D), jnp.float32),
)


def _fin_body(deg_ref, a_ref, y_ref, b_ref, o_ref):
    dinv = _dinv_col(deg_ref)
    o_ref[...] = (a_ref[0] + a_ref[1] + y_ref[...]) * dinv + b_ref[...]


_fin = pl.pallas_call(
    _fin_body,
    grid=(_GRID,),
    in_specs=[
        pl.BlockSpec((2, _BM), lambda i: (0, i)),
        pl.BlockSpec((2, _BM, D), lambda i: (0, i, 0)),
        pl.BlockSpec((_BM, D), lambda i: (i, 0)),
        pl.BlockSpec((1, D), lambda i: (0, 0)),
    ],
    out_specs=pl.BlockSpec((_BM, D), lambda i: (i, 0)),
    out_shape=jax.ShapeDtypeStruct((NP, D), jnp.float32),
)


def kernel(x, edge_index, W1, b1, W2, b2):
    src, dst = edge_index[0], edge_index[1]
    pad_idx = (N + (jnp.arange(EP - E, dtype=jnp.int32) % (NP - N)))
    src_p = jnp.concatenate([src, pad_idx]).reshape(NW, EPW)
    dst_p = jnp.concatenate([dst, pad_idx]).reshape(NW, NBLK, K)

    x_p = jnp.pad(x, ((0, NP - N), (0, 0)))
    b1r = b1.reshape(1, D)
    b2r = b2.reshape(1, D)

    zeros1 = jnp.zeros((RPT,), jnp.float32)
    ones_k = jnp.ones((K,), jnp.float32)
    zrows = jnp.zeros((RPT, D), jnp.float32)

    deg = _deg_kernel()(dst_p, zeros1, ones_k)
    y1 = _mm_scale(deg, x_p, W1)
    a1 = _edge_kernel()(y1, src_p, dst_p, zrows)
    y2 = _mid(deg, a1, y1, b1r, W2)
    a2 = _edge_kernel()(y2, src_p, dst_p, zrows)
    out = _fin(deg, a2, y2, b2r)
    return out[:N]
